# Initial kernel scaffold; baseline (speedup 1.0000x reference)
import functools

import jax
import jax.numpy as jnp
from jax import lax
from jax.experimental import pallas as pl
from jax.experimental.pallas import tpu as pltpu
from jax.experimental.pallas import tpu_sc as plsc

N = 10000
E = 320000
D = 128
DE = 16
H = 128

NC = 2
NS = 16
L = 16

ES = E // NS
CK = 80
NCH = ES // CK
NCH2 = NCH // NC
RPT = N // NS
EPR = 128


def _pre_body(x_ref, ea_ref, w_ref, asr_ref, adr_ref, wer_ref, aer_ref,
              h_ref, as_ref, ad_ref, ae_ref, sh_ref):
    h = jnp.dot(x_ref[...], w_ref[...], preferred_element_type=jnp.float32)
    h_ref[...] = h
    a_s = jnp.dot(h, asr_ref[...], preferred_element_type=jnp.float32)
    a_d = jnp.dot(h, adr_ref[...], preferred_element_type=jnp.float32)
    as_ref[...] = a_s
    ad_ref[...] = a_d
    u = jnp.dot(wer_ref[...], aer_ref[...], preferred_element_type=jnp.float32)
    urep = jnp.concatenate([u] * EPR, axis=0)
    row = lax.broadcasted_iota(jnp.int32, (DE * EPR, EPR), 0)
    col = lax.broadcasted_iota(jnp.int32, (DE * EPR, EPR), 1)
    u3 = jnp.where((row // DE) == col, urep, 0.0)
    ae = jnp.dot(ea_ref[...], u3, preferred_element_type=jnp.float32)
    ae_ref[...] = ae
    sh = jnp.maximum(jnp.max(a_s) + jnp.max(a_d) + jnp.max(ae), 0.0)
    sh_ref[...] = jnp.zeros((1, 1), jnp.float32) + sh


_pre = pl.pallas_call(
    _pre_body,
    out_shape=[
        jax.ShapeDtypeStruct((N, H), jnp.float32),
        jax.ShapeDtypeStruct((N, 1), jnp.float32),
        jax.ShapeDtypeStruct((N, 1), jnp.float32),
        jax.ShapeDtypeStruct((E // EPR, EPR), jnp.float32),
        jax.ShapeDtypeStruct((1, 1), jnp.float32),
    ],
)


def _sc_body(src_h, dst_h, ae_h, as_h, ad_h, sh_h, h_h, out_h,
             asv, adv, srcv, dstv, aev, exv, dnv, shv, coefv, rowbuf,
             sacc, sden, sem):
    c = lax.axis_index("c")
    s = lax.axis_index("s")
    zero = jnp.zeros((L,), jnp.float32)

    pltpu.sync_copy(as_h, asv)
    pltpu.sync_copy(ad_h, adv)
    pltpu.sync_copy(src_h.at[s], srcv)
    pltpu.sync_copy(dst_h.at[s], dstv)
    pltpu.sync_copy(ae_h.at[s], aev)
    pltpu.sync_copy(sh_h, shv)

    def zrow(r, carry):
        for q in range(H // L):
            rowbuf[r, pl.ds(q * L, L)] = zero
        return carry
    lax.fori_loop(0, CK, zrow, 0)
    base = s * RPT
    for w in range(RPT // CK):
        pltpu.sync_copy(rowbuf, sacc.at[pl.ds(base + w * CK, CK)])
    rem = RPT - (RPT // CK) * CK
    pltpu.sync_copy(rowbuf.at[pl.ds(0, rem)],
                    sacc.at[pl.ds(base + (RPT // CK) * CK, rem)])

    def zden(i, carry):
        dnv[pl.ds(i * L, L)] = zero
        return carry
    lax.fori_loop(0, N // L, zden, 0)

    @pl.when(s == 0)
    def _():
        pltpu.sync_copy(dnv, sden)

    plsc.subcore_barrier()

    shift = shv[...]

    def p1(j, carry):
        for k in range(CK // L):
            sl = pl.ds(k * L, L)
            sv = srcv[j, sl]
            dv = dstv[j, sl]
            av = plsc.load_gather(asv, [sv])
            bv = plsc.load_gather(adv, [dv])
            al = av + bv + aev[j, sl]
            al = jnp.where(al >= 0.0, al, al * 0.2)
            exv[j, sl] = jnp.exp(al - shift)
        pltpu.sync_copy(exv.at[j], sden.at[dstv.at[j]], add=True)
        return carry
    lax.fori_loop(0, NCH, p1, 0)

    plsc.subcore_barrier()
    pltpu.sync_copy(sden, dnv)

    def p2(j2, carry):
        j = c * NCH2 + j2
        cp = pltpu.async_copy(h_h.at[srcv.at[j]], rowbuf, sem)
        for k in range(CK // L):
            sl = pl.ds(k * L, L)
            dv = dstv[j, sl]
            dn = plsc.load_gather(dnv, [dv])
            coefv[sl] = exv[j, sl] / (dn + 1e-16)
        cp.wait()

        def rowfn(r, rcarry):
            rv = jnp.zeros((L,), jnp.int32) + r
            cs = plsc.load_gather(coefv, [rv])
            for q in range(H // L):
                ql = pl.ds(q * L, L)
                rowbuf[r, ql] = rowbuf[r, ql] * cs
            return rcarry
        lax.fori_loop(0, CK, rowfn, 0)
        pltpu.sync_copy(rowbuf, sacc.at[dstv.at[j]], add=True)
        return carry
    lax.fori_loop(0, NCH2, p2, 0)

    plsc.subcore_barrier()

    for w in range(RPT // CK):
        pltpu.sync_copy(sacc.at[pl.ds(base + w * CK, CK)], rowbuf)
        pltpu.sync_copy(rowbuf, out_h.at[c, pl.ds(base + w * CK, CK)])
    pltpu.sync_copy(sacc.at[pl.ds(base + (RPT // CK) * CK, rem)],
                    rowbuf.at[pl.ds(0, rem)])
    pltpu.sync_copy(rowbuf.at[pl.ds(0, rem)],
                    out_h.at[c, pl.ds(base + (RPT // CK) * CK, rem)])


_sc = pl.kernel(
    _sc_body,
    out_type=jax.ShapeDtypeStruct((NC, N, H), jnp.float32),
    mesh=plsc.VectorSubcoreMesh(core_axis_name="c", subcore_axis_name="s"),
    scratch_types=[
        pltpu.VMEM((N,), jnp.float32),
        pltpu.VMEM((N,), jnp.float32),
        pltpu.VMEM((NCH, CK), jnp.int32),
        pltpu.VMEM((NCH, CK), jnp.int32),
        pltpu.VMEM((NCH, CK), jnp.float32),
        pltpu.VMEM((NCH, CK), jnp.float32),
        pltpu.VMEM((N,), jnp.float32),
        pltpu.VMEM((L,), jnp.float32),
        pltpu.VMEM((CK,), jnp.float32),
        pltpu.VMEM((CK, H), jnp.float32),
        pltpu.VMEM_SHARED((N, H), jnp.float32),
        pltpu.VMEM_SHARED((N,), jnp.float32),
        pltpu.SemaphoreType.DMA,
    ],
)


def _post_body(p_ref, b_ref, wl_ref, bl_ref, o_ref):
    t = p_ref[0] + p_ref[1] + b_ref[...]
    t = jnp.maximum(t, 0.0)
    o_ref[...] = (jnp.dot(t, wl_ref[...], preferred_element_type=jnp.float32)
                  + bl_ref[...])


_post = pl.pallas_call(
    _post_body,
    out_shape=jax.ShapeDtypeStruct((N, 1), jnp.float32),
)


def kernel(node_static_features, edge_static_features, edge_index, W,
           att_src, att_dst, W_edge, att_edge, bias, W_lin, b_lin):
    x = node_static_features.astype(jnp.float32)
    ea3 = edge_static_features.astype(jnp.float32).reshape(E // EPR, EPR * DE)
    h, a_s, a_d, ae2, sh = _pre(
        x, ea3, W, att_src.reshape(H, 1), att_dst.reshape(H, 1),
        W_edge, att_edge.reshape(H, 1))
    src3 = edge_index[:, 0].reshape(NS, NCH, CK)
    dst3 = edge_index[:, 1].reshape(NS, NCH, CK)
    ae3 = ae2.reshape(NS, NCH, CK)
    sh16 = jnp.broadcast_to(sh.reshape(()), (L,))
    parts = _sc(src3, dst3, ae3, a_s.reshape(N), a_d.reshape(N), sh16, h)
    return _post(parts, bias.reshape(1, H), W_lin, b_lin.reshape(1, 1))


# speedup vs baseline: 17.3310x; 17.3310x over previous
import jax
import jax.numpy as jnp
from jax import lax
from jax.experimental import pallas as pl
from jax.experimental.pallas import tpu as pltpu
from jax.experimental.pallas import tpu_sc as plsc

N = 10000
E = 320000
D = 128
DE = 16
H = 128

NC = 2
NS = 16
L = 16

CK = 80
NCH = E // CK // NS
NCH2 = NCH // NC
G = 5
RPT = 624
REM = N - RPT * NS
EPR = 128


def _pre_body(x_ref, ea_ref, w_ref, asr_ref, adr_ref, wer_ref, aer_ref,
              h_ref, as_ref, ad_ref, ae_ref, sh_ref):
    h = jnp.dot(x_ref[...], w_ref[...], preferred_element_type=jnp.float32)
    h_ref[...] = h
    a_s = jnp.dot(h, asr_ref[...], preferred_element_type=jnp.float32)
    a_d = jnp.dot(h, adr_ref[...], preferred_element_type=jnp.float32)
    as_ref[...] = a_s
    ad_ref[...] = a_d
    u = jnp.dot(wer_ref[...], aer_ref[...], preferred_element_type=jnp.float32)
    urep = jnp.concatenate([u] * EPR, axis=0)
    row = lax.broadcasted_iota(jnp.int32, (DE * EPR, EPR), 0)
    col = lax.broadcasted_iota(jnp.int32, (DE * EPR, EPR), 1)
    u3 = jnp.where((row // DE) == col, urep, 0.0)
    ae = jnp.dot(ea_ref[...], u3, preferred_element_type=jnp.float32)
    ae_ref[...] = ae
    sh = jnp.maximum(jnp.max(a_s) + jnp.max(a_d) + jnp.max(ae), 0.0)
    sh_ref[...] = jnp.zeros((1, 1), jnp.float32) + sh


_pre = pl.pallas_call(
    _pre_body,
    out_shape=[
        jax.ShapeDtypeStruct((N, H), jnp.float32),
        jax.ShapeDtypeStruct((N, 1), jnp.float32),
        jax.ShapeDtypeStruct((N, 1), jnp.float32),
        jax.ShapeDtypeStruct((E // EPR, EPR), jnp.float32),
        jax.ShapeDtypeStruct((1, 1), jnp.float32),
    ],
)


def _edge_vectors(pkb, t, k, asv, adv, shift):
    sl = pl.ds(k * L, L)
    sv = pkb[t, 0, sl]
    dv = pkb[t, 1, sl]
    ae = plsc.bitcast(pkb[t, 2, sl], jnp.float32)
    av = plsc.load_gather(asv, [sv])
    bv = plsc.load_gather(adv, [dv])
    al = av + bv + ae
    al = jnp.where(al >= 0.0, al, al * 0.2)
    return dv, jnp.exp(al - shift)


def _sc_body(pk_h, as_h, ad_h, sh_h, h_h, out_h,
             asv, adv, dnv, shv, pkb, exc, coefv, rowbuf, sacc, sden, sem):
    c = lax.axis_index("c")
    s = lax.axis_index("s")
    zero = jnp.zeros((L,), jnp.float32)

    pltpu.sync_copy(as_h, asv)
    pltpu.sync_copy(ad_h, adv)
    pltpu.sync_copy(sh_h, shv)

    def zrow(r, carry):
        for q in range(H // L):
            rowbuf[r, pl.ds(q * L, L)] = zero
        return carry
    lax.fori_loop(0, CK, zrow, 0)
    base = s * RPT
    nfull = RPT // CK
    rem = RPT - nfull * CK
    for w in range(nfull):
        pltpu.sync_copy(rowbuf, sacc.at[pl.ds(base + w * CK, CK)])
    pltpu.sync_copy(rowbuf.at[pl.ds(0, rem)],
                    sacc.at[pl.ds(base + nfull * CK, rem)])

    def zden(i, carry):
        dnv[pl.ds(i * L, L)] = zero
        return carry
    lax.fori_loop(0, N // L, zden, 0)

    @pl.when(s == 0)
    def _():
        pltpu.sync_copy(rowbuf.at[pl.ds(0, REM)],
                        sacc.at[pl.ds(RPT * NS, REM)])
        pltpu.sync_copy(dnv, sden)

    plsc.subcore_barrier()

    shift = shv[...]

    def p1(g, carry):
        pltpu.sync_copy(pk_h.at[pl.ds(s * NCH + g * G, G)], pkb)
        for t in range(G):
            for k in range(CK // L):
                _, ex = _edge_vectors(pkb, t, k, asv, adv, shift)
                exc[pl.ds(k * L, L)] = ex
            pltpu.sync_copy(exc, sden.at[pkb.at[t, 1]], add=True)
        return carry
    lax.fori_loop(0, NCH // G, p1, 0)

    plsc.subcore_barrier()
    pltpu.sync_copy(sden, dnv)

    def p2(g, carry):
        pltpu.sync_copy(pk_h.at[pl.ds(s * NCH + c * NCH2 + g * G, G)], pkb)
        for t in range(G):
            cp = pltpu.async_copy(h_h.at[pkb.at[t, 0]], rowbuf, sem)
            for k in range(CK // L):
                dv, ex = _edge_vectors(pkb, t, k, asv, adv, shift)
                dn = plsc.load_gather(dnv, [dv])
                coefv[pl.ds(k * L, L)] = ex / (dn + 1e-16)
            cp.wait()

            def rowfn(r, rcarry):
                rv = jnp.zeros((L,), jnp.int32) + r
                cs = plsc.load_gather(coefv, [rv])
                for q in range(H // L):
                    ql = pl.ds(q * L, L)
                    rowbuf[r, ql] = rowbuf[r, ql] * cs
                return rcarry
            lax.fori_loop(0, CK, rowfn, 0)
            pltpu.sync_copy(rowbuf, sacc.at[pkb.at[t, 1]], add=True)
        return carry
    lax.fori_loop(0, NCH2 // G, p2, 0)

    plsc.subcore_barrier()

    for w in range(nfull):
        pltpu.sync_copy(sacc.at[pl.ds(base + w * CK, CK)], rowbuf)
        pltpu.sync_copy(rowbuf, out_h.at[c, pl.ds(base + w * CK, CK)])
    pltpu.sync_copy(sacc.at[pl.ds(base + nfull * CK, rem)],
                    rowbuf.at[pl.ds(0, rem)])
    pltpu.sync_copy(rowbuf.at[pl.ds(0, rem)],
                    out_h.at[c, pl.ds(base + nfull * CK, rem)])

    @pl.when(s == 0)
    def _():
        pltpu.sync_copy(sacc.at[pl.ds(RPT * NS, REM)], rowbuf.at[pl.ds(0, REM)])
        pltpu.sync_copy(rowbuf.at[pl.ds(0, REM)],
                        out_h.at[c, pl.ds(RPT * NS, REM)])


_sc = pl.kernel(
    _sc_body,
    out_type=jax.ShapeDtypeStruct((NC, N, H), jnp.float32),
    mesh=plsc.VectorSubcoreMesh(core_axis_name="c", subcore_axis_name="s"),
    compiler_params=pltpu.CompilerParams(needs_layout_passes=False,
                                         use_tc_tiling_on_sc=False),
    scratch_types=[
        pltpu.VMEM((N,), jnp.float32),
        pltpu.VMEM((N,), jnp.float32),
        pltpu.VMEM((N,), jnp.float32),
        pltpu.VMEM((L,), jnp.float32),
        pltpu.VMEM((G, 3, CK), jnp.int32),
        pltpu.VMEM((CK,), jnp.float32),
        pltpu.VMEM((CK,), jnp.float32),
        pltpu.VMEM((CK, H), jnp.float32),
        pltpu.VMEM_SHARED((N, H), jnp.float32),
        pltpu.VMEM_SHARED((N,), jnp.float32),
        pltpu.SemaphoreType.DMA,
    ],
)


def _post_body(p_ref, b_ref, wl_ref, bl_ref, o_ref):
    t = p_ref[0] + p_ref[1] + b_ref[...]
    t = jnp.maximum(t, 0.0)
    o_ref[...] = (jnp.dot(t, wl_ref[...], preferred_element_type=jnp.float32)
                  + bl_ref[...])


_post = pl.pallas_call(
    _post_body,
    out_shape=jax.ShapeDtypeStruct((N, 1), jnp.float32),
)


def kernel(node_static_features, edge_static_features, edge_index, W,
           att_src, att_dst, W_edge, att_edge, bias, W_lin, b_lin):
    x = node_static_features.astype(jnp.float32)
    ea3 = edge_static_features.astype(jnp.float32).reshape(E // EPR, EPR * DE)
    h, a_s, a_d, ae2, sh = _pre(
        x, ea3, W, att_src.reshape(H, 1), att_dst.reshape(H, 1),
        W_edge, att_edge.reshape(H, 1))
    src4 = edge_index[:, 0].reshape(E // CK, 1, CK)
    dst4 = edge_index[:, 1].reshape(E // CK, 1, CK)
    ae4 = lax.bitcast_convert_type(ae2.reshape(E), jnp.int32)
    pk = jnp.concatenate([src4, dst4, ae4.reshape(E // CK, 1, CK)], axis=1)
    sh16 = jnp.broadcast_to(sh.reshape(()), (L,))
    parts = _sc(pk, a_s.reshape(N), a_d.reshape(N), sh16, h)
    return _post(parts, bias.reshape(1, H), W_lin, b_lin.reshape(1, 1))
